# Initial kernel scaffold; baseline (speedup 1.0000x reference)
import functools

import jax
import jax.numpy as jnp
from jax import lax
from jax.experimental import pallas as pl
from jax.experimental.pallas import tpu as pltpu
from jax.experimental.pallas import tpu_sc as plsc

NC = 2
NS = 16
NW = NC * NS
LANES = 16
CHUNK = 128
HIDP = 16


def _deg_body(dst_hbm, zeros_hbm, out_hbm, idx_v, ones_v, deg_sh):
    n_pad = deg_sh.shape[0]
    rows = n_pad // NS
    cpw = idx_v.shape[0]
    cid = lax.axis_index("c")
    sid = lax.axis_index("s")
    wid = sid * NC + cid
    pltpu.sync_copy(dst_hbm.at[wid], idx_v)
    pltpu.sync_copy(zeros_hbm.at[pl.ds(sid * rows, rows)],
                    deg_sh.at[pl.ds(sid * rows, rows)])
    for i in range(CHUNK // LANES):
        ones_v[pl.ds(i * LANES, LANES)] = jnp.ones((LANES,), jnp.float32)
    plsc.subcore_barrier()

    def body(j, carry):
        pltpu.sync_copy(ones_v, deg_sh.at[idx_v.at[j]], add=True)
        return carry

    lax.fori_loop(0, cpw, body, 0)
    plsc.subcore_barrier()
    pltpu.sync_copy(deg_sh.at[pl.ds(sid * rows, rows)],
                    out_hbm.at[cid, pl.ds(sid * rows, rows)])


def _edge_body(src_hbm, dst_hbm, table_hbm, zeros_hbm, out_hbm,
               sidx_v, didx_v, rows_v, table_sh, acc_sh, sem):
    n_pad = acc_sh.shape[0]
    rows = n_pad // NS
    cpw = sidx_v.shape[0]
    cid = lax.axis_index("c")
    sid = lax.axis_index("s")
    wid = sid * NC + cid
    pltpu.sync_copy(src_hbm.at[wid], sidx_v)
    pltpu.sync_copy(dst_hbm.at[wid], didx_v)
    pltpu.sync_copy(table_hbm.at[pl.ds(sid * rows, rows)],
                    table_sh.at[pl.ds(sid * rows, rows)])
    pltpu.sync_copy(zeros_hbm.at[pl.ds(sid * rows, rows)],
                    acc_sh.at[pl.ds(sid * rows, rows)])
    plsc.subcore_barrier()

    def body(j, carry):
        pltpu.async_copy(table_sh.at[sidx_v.at[j]], rows_v, sem).wait()
        pltpu.sync_copy(rows_v, acc_sh.at[didx_v.at[j]], add=True)
        return carry

    lax.fori_loop(0, cpw, body, 0)
    plsc.subcore_barrier()
    pltpu.sync_copy(acc_sh.at[pl.ds(sid * rows, rows)],
                    out_hbm.at[cid, pl.ds(sid * rows, rows)])


def _tc1_body(x_ref, w_ref, degp_ref, hn_ref, dinv_ref):
    deg = degp_ref[0] + degp_ref[1] + 1.0
    dinv = lax.rsqrt(deg)
    h = jnp.dot(x_ref[...], w_ref[...], preferred_element_type=jnp.float32)
    hn_ref[...] = h * dinv
    dinv_ref[...] = dinv


def _tc2_body(p_ref, hn1_ref, dinv_ref, b1_ref, w2_ref, hn2_ref):
    s = p_ref[0] + p_ref[1] + hn1_ref[...]
    h1 = jnp.maximum(dinv_ref[...] * s + b1_ref[...], 0.0)
    hn2_ref[...] = (
        jnp.dot(h1, w2_ref[...], preferred_element_type=jnp.float32)
        * dinv_ref[...])


def _tc3_body(p_ref, hn2_ref, dinv_ref, b2_ref, out_ref):
    s = p_ref[0] + p_ref[1] + hn2_ref[...]
    out_ref[...] = dinv_ref[...] * s + b2_ref[...]


@functools.partial(jax.jit, static_argnames=("n_pad", "cpw"))
def _run(x_p, srcp, dstp, W1, b1r, w2p, b2r, n_pad, cpw):
    f32 = jnp.float32
    mesh = plsc.VectorSubcoreMesh(core_axis_name="c", subcore_axis_name="s")
    zeros1 = jnp.zeros((n_pad,), f32)
    zeros2 = jnp.zeros((n_pad, HIDP), f32)

    deg_call = pl.kernel(
        _deg_body,
        out_type=jax.ShapeDtypeStruct((NC, n_pad), f32),
        mesh=mesh,
        scratch_types=[
            pltpu.VMEM((cpw, CHUNK), jnp.int32),
            pltpu.VMEM((CHUNK,), f32),
            pltpu.VMEM_SHARED((n_pad,), f32),
        ],
    )
    edge_call = pl.kernel(
        _edge_body,
        out_type=jax.ShapeDtypeStruct((NC, n_pad, HIDP), f32),
        mesh=mesh,
        scratch_types=[
            pltpu.VMEM((cpw, CHUNK), jnp.int32),
            pltpu.VMEM((cpw, CHUNK), jnp.int32),
            pltpu.VMEM((CHUNK, HIDP), f32),
            pltpu.VMEM_SHARED((n_pad, HIDP), f32),
            pltpu.VMEM_SHARED((n_pad, HIDP), f32),
            pltpu.SemaphoreType.DMA,
        ],
    )

    degp = deg_call(dstp, zeros1)
    hn1, dinv = pl.pallas_call(
        _tc1_body,
        out_shape=(jax.ShapeDtypeStruct((n_pad, HIDP), f32),
                   jax.ShapeDtypeStruct((n_pad, 1), f32)),
    )(x_p, W1, degp.reshape(NC, n_pad, 1))
    p1 = edge_call(srcp, dstp, hn1, zeros2)
    hn2 = pl.pallas_call(
        _tc2_body,
        out_shape=jax.ShapeDtypeStruct((n_pad, HIDP), f32),
    )(p1, hn1, dinv, b1r, w2p)
    p2 = edge_call(srcp, dstp, hn2, zeros2)
    outf = pl.pallas_call(
        _tc3_body,
        out_shape=jax.ShapeDtypeStruct((n_pad, HIDP), f32),
    )(p2, hn2, dinv, b2r)
    return outf


def kernel(x, edge_index, W1, b1, W2, b2):
    f32 = jnp.float32
    n, in_ch = x.shape
    hid = W1.shape[1]
    out_ch = W2.shape[1]
    n_edges = edge_index.shape[1]

    block = NW * CHUNK
    cpw = -(-n_edges // block)
    e_pad = cpw * block
    n_pad = -(-n // (NS * 8)) * (NS * 8)
    dummy = n

    ei = edge_index.astype(jnp.int32)
    pad = jnp.full((e_pad - n_edges,), dummy, jnp.int32)
    srcp = jnp.concatenate([ei[0], pad]).reshape(NW, cpw, CHUNK)
    dstp = jnp.concatenate([ei[1], pad]).reshape(NW, cpw, CHUNK)
    x_p = jnp.pad(x.astype(f32), ((0, n_pad - n), (0, 0)))
    b1r = b1.reshape(1, hid).astype(f32)
    w2p = jnp.pad(W2.astype(f32), ((0, 0), (0, HIDP - out_ch)))
    b2r = jnp.pad(b2.reshape(1, out_ch).astype(f32),
                  ((0, 0), (0, HIDP - out_ch)))

    outf = _run(x_p, srcp, dstp, W1.astype(f32), b1r, w2p, b2r,
                n_pad=n_pad, cpw=cpw)
    return outf[:n, :out_ch]


# speedup vs baseline: 26.5019x; 26.5019x over previous
import functools

import jax
import jax.numpy as jnp
from jax import lax
from jax.experimental import pallas as pl
from jax.experimental.pallas import tpu as pltpu
from jax.experimental.pallas import tpu_sc as plsc

NC = 2
NS = 16
NW = NC * NS
LANES = 16
CHUNK = 128
IB = 8
HIDP = 16


def _deg_body(dst_hbm, out_hbm, idx_v, ones_v, stage_v, deg_sh):
    n_pad = deg_sh.shape[0]
    rows = n_pad // NS
    n_b = dst_hbm.shape[1] // IB
    cid = lax.axis_index("c")
    sid = lax.axis_index("s")
    wid = sid * NC + cid

    def obody(i, carry):
        ones_v[i] = jnp.ones((LANES,), jnp.float32)
        return carry

    lax.fori_loop(0, CHUNK, obody, 0)

    def zbody(i, carry):
        stage_v[i] = jnp.zeros((LANES,), jnp.float32)
        return carry

    lax.fori_loop(0, rows, zbody, 0)
    pltpu.sync_copy(stage_v, deg_sh.at[pl.ds(sid * rows, rows)])
    plsc.subcore_barrier()

    def outer(bi, carry):
        pltpu.sync_copy(dst_hbm.at[wid, pl.ds(bi * IB, IB)], idx_v)

        def body(j, c2):
            pltpu.sync_copy(ones_v, deg_sh.at[idx_v.at[j]], add=True)
            return c2

        return lax.fori_loop(0, IB, body, carry)

    lax.fori_loop(0, n_b, outer, 0)
    plsc.subcore_barrier()
    pltpu.sync_copy(deg_sh.at[pl.ds(sid * rows, rows)], stage_v)
    pltpu.sync_copy(stage_v,
                    out_hbm.at[pl.ds(cid * n_pad + sid * rows, rows)])


def _edge_body(src_hbm, dst_hbm, table_hbm, out_hbm,
               sidx_v, didx_v, rows_v, stage_v, acc_sh, sem):
    n_pad = acc_sh.shape[0]
    rows = n_pad // NS
    n_b = src_hbm.shape[1] // IB
    cid = lax.axis_index("c")
    sid = lax.axis_index("s")
    wid = sid * NC + cid

    def zbody(i, carry):
        stage_v[i] = jnp.zeros((LANES,), jnp.float32)
        return carry

    lax.fori_loop(0, rows, zbody, 0)
    pltpu.sync_copy(stage_v, acc_sh.at[pl.ds(sid * rows, rows)])
    plsc.subcore_barrier()

    def outer(bi, carry):
        pltpu.sync_copy(src_hbm.at[wid, pl.ds(bi * IB, IB)], sidx_v)
        pltpu.sync_copy(dst_hbm.at[wid, pl.ds(bi * IB, IB)], didx_v)

        def body(j, c2):
            pltpu.async_copy(table_hbm.at[sidx_v.at[j]], rows_v, sem).wait()
            pltpu.sync_copy(rows_v, acc_sh.at[didx_v.at[j]], add=True)
            return c2

        return lax.fori_loop(0, IB, body, carry)

    lax.fori_loop(0, n_b, outer, 0)
    plsc.subcore_barrier()
    pltpu.sync_copy(acc_sh.at[pl.ds(sid * rows, rows)], stage_v)
    pltpu.sync_copy(stage_v,
                    out_hbm.at[pl.ds(cid * n_pad + sid * rows, rows)])


def _tc1_body(x_ref, w_ref, degp_ref, hn_ref, dinv_ref):
    deg = degp_ref[0] + degp_ref[1] + 1.0
    dinv = lax.rsqrt(deg)
    h = jnp.dot(x_ref[...], w_ref[...], preferred_element_type=jnp.float32)
    hn_ref[...] = h * dinv
    dinv_ref[...] = dinv


def _tc2_body(p_ref, hn1_ref, dinv_ref, b1_ref, w2_ref, hn2_ref):
    s = p_ref[0] + p_ref[1] + hn1_ref[...]
    h1 = jnp.maximum(dinv_ref[...] * s + b1_ref[...], 0.0)
    hn2_ref[...] = (
        jnp.dot(h1, w2_ref[...], preferred_element_type=jnp.float32)
        * dinv_ref[...])


def _tc3_body(p_ref, hn2_ref, dinv_ref, b2_ref, out_ref):
    s = p_ref[0] + p_ref[1] + hn2_ref[...]
    out_ref[...] = dinv_ref[...] * s + b2_ref[...]


@functools.partial(jax.jit, static_argnames=("n_pad", "cpw"))
def _run(x_p, srcp, dstp, W1, b1r, w2p, b2r, n_pad, cpw):
    f32 = jnp.float32
    mesh = plsc.VectorSubcoreMesh(core_axis_name="c", subcore_axis_name="s")
    rows = n_pad // NS

    sc_params = pltpu.CompilerParams(use_tc_tiling_on_sc=False)
    deg_call = pl.kernel(
        _deg_body,
        out_type=jax.ShapeDtypeStruct((NC * n_pad, HIDP), f32),
        mesh=mesh,
        compiler_params=sc_params,
        scratch_types=[
            pltpu.VMEM((IB, CHUNK), jnp.int32),
            pltpu.VMEM((CHUNK, HIDP), f32),
            pltpu.VMEM((rows, HIDP), f32),
            pltpu.VMEM_SHARED((n_pad, HIDP), f32),
        ],
    )
    edge_call = pl.kernel(
        _edge_body,
        out_type=jax.ShapeDtypeStruct((NC * n_pad, HIDP), f32),
        mesh=mesh,
        compiler_params=sc_params,
        scratch_types=[
            pltpu.VMEM((IB, CHUNK), jnp.int32),
            pltpu.VMEM((IB, CHUNK), jnp.int32),
            pltpu.VMEM((CHUNK, HIDP), f32),
            pltpu.VMEM((rows, HIDP), f32),
            pltpu.VMEM_SHARED((n_pad, HIDP), f32),
            pltpu.SemaphoreType.DMA,
        ],
    )

    degp = deg_call(dstp)
    hn1, dinv = pl.pallas_call(
        _tc1_body,
        out_shape=(jax.ShapeDtypeStruct((n_pad, HIDP), f32),
                   jax.ShapeDtypeStruct((n_pad, HIDP), f32)),
    )(x_p, W1, degp.reshape(NC, n_pad, HIDP))
    p1 = edge_call(srcp, dstp, hn1)
    hn2 = pl.pallas_call(
        _tc2_body,
        out_shape=jax.ShapeDtypeStruct((n_pad, HIDP), f32),
    )(p1.reshape(NC, n_pad, HIDP), hn1, dinv, b1r, w2p)
    p2 = edge_call(srcp, dstp, hn2)
    outf = pl.pallas_call(
        _tc3_body,
        out_shape=jax.ShapeDtypeStruct((n_pad, HIDP), f32),
    )(p2.reshape(NC, n_pad, HIDP), hn2, dinv, b2r)
    return outf


def kernel(x, edge_index, W1, b1, W2, b2):
    f32 = jnp.float32
    n, in_ch = x.shape
    hid = W1.shape[1]
    out_ch = W2.shape[1]
    n_edges = edge_index.shape[1]

    block = NW * CHUNK * IB
    cpw = IB * (-(-n_edges // block))
    e_pad = cpw * NW * CHUNK
    n_pad = -(-n // (NS * 8)) * (NS * 8)
    dummy = n

    ei = edge_index.astype(jnp.int32)
    pad = jnp.full((e_pad - n_edges,), dummy, jnp.int32)
    srcp = jnp.concatenate([ei[0], pad]).reshape(NW, cpw, CHUNK)
    dstp = jnp.concatenate([ei[1], pad]).reshape(NW, cpw, CHUNK)
    x_p = jnp.pad(x.astype(f32), ((0, n_pad - n), (0, 0)))
    b1r = b1.reshape(1, hid).astype(f32)
    w2p = jnp.pad(W2.astype(f32), ((0, 0), (0, HIDP - out_ch)))
    b2r = jnp.pad(b2.reshape(1, out_ch).astype(f32),
                  ((0, 0), (0, HIDP - out_ch)))

    outf = _run(x_p, srcp, dstp, W1.astype(f32), b1r, w2p, b2r,
                n_pad=n_pad, cpw=cpw)
    return outf[:n, :out_ch]


# speedup vs baseline: 33.5309x; 1.2652x over previous
import functools

import jax
import jax.numpy as jnp
from jax import lax
from jax.experimental import pallas as pl
from jax.experimental.pallas import tpu as pltpu
from jax.experimental.pallas import tpu_sc as plsc

NC = 2
NS = 16
NW = NC * NS
LANES = 16
CHUNK = 128
IB = 8
HIDP = 16


def _deg_body(dst_hbm, out_hbm, idx_v, ones_v, stage_v, deg_sh, ssem):
    n_pad = deg_sh.shape[0]
    rows = n_pad // NS
    n_b = dst_hbm.shape[1] // IB
    cid = lax.axis_index("c")
    sid = lax.axis_index("s")
    wid = sid * NC + cid

    def obody(i, carry):
        ones_v[i] = jnp.ones((LANES,), jnp.float32)
        return carry

    lax.fori_loop(0, CHUNK, obody, 0)

    def zbody(i, carry):
        stage_v[i] = jnp.zeros((LANES,), jnp.float32)
        return carry

    lax.fori_loop(0, rows, zbody, 0)
    pltpu.sync_copy(stage_v, deg_sh.at[pl.ds(sid * rows, rows)])
    plsc.subcore_barrier()

    def outer(bi, carry):
        pltpu.sync_copy(dst_hbm.at[wid, pl.ds(bi * IB, IB)], idx_v)
        descs = [
            pltpu.async_copy(ones_v, deg_sh.at[idx_v.at[j]], ssem, add=True)
            for j in range(IB)
        ]
        for d in descs:
            d.wait()
        return carry

    lax.fori_loop(0, n_b, outer, 0)
    plsc.subcore_barrier()
    pltpu.sync_copy(deg_sh.at[pl.ds(sid * rows, rows)], stage_v)
    pltpu.sync_copy(stage_v,
                    out_hbm.at[pl.ds(cid * n_pad + sid * rows, rows)])


def _edge_body(src_hbm, dst_hbm, table_hbm, out_hbm,
               sidx_v, didx_v, rows_v, stage_v, acc_sh, gsems, ssem):
    n_pad = acc_sh.shape[0]
    rows = n_pad // NS
    n_b = src_hbm.shape[1] // IB
    cid = lax.axis_index("c")
    sid = lax.axis_index("s")
    wid = sid * NC + cid

    def zbody(i, carry):
        stage_v[i] = jnp.zeros((LANES,), jnp.float32)
        return carry

    lax.fori_loop(0, rows, zbody, 0)
    pltpu.sync_copy(stage_v, acc_sh.at[pl.ds(sid * rows, rows)])
    plsc.subcore_barrier()

    def outer(bi, carry):
        pltpu.sync_copy(src_hbm.at[wid, pl.ds(bi * IB, IB)], sidx_v)
        pltpu.sync_copy(dst_hbm.at[wid, pl.ds(bi * IB, IB)], didx_v)
        gd = [
            pltpu.async_copy(table_hbm.at[sidx_v.at[j]], rows_v.at[j],
                             gsems.at[j])
            for j in range(IB)
        ]
        sd = []
        for j in range(IB):
            gd[j].wait()
            sd.append(
                pltpu.async_copy(rows_v.at[j], acc_sh.at[didx_v.at[j]],
                                 ssem, add=True))
        for d in sd:
            d.wait()
        return carry

    lax.fori_loop(0, n_b, outer, 0)
    plsc.subcore_barrier()
    pltpu.sync_copy(acc_sh.at[pl.ds(sid * rows, rows)], stage_v)
    pltpu.sync_copy(stage_v,
                    out_hbm.at[pl.ds(cid * n_pad + sid * rows, rows)])


def _tc1_body(x_ref, w_ref, degp_ref, hn_ref, dinv_ref):
    deg = degp_ref[0] + degp_ref[1] + 1.0
    dinv = lax.rsqrt(deg)
    h = jnp.dot(x_ref[...], w_ref[...], preferred_element_type=jnp.float32)
    hn_ref[...] = h * dinv
    dinv_ref[...] = dinv


def _tc2_body(p_ref, hn1_ref, dinv_ref, b1_ref, w2_ref, hn2_ref):
    s = p_ref[0] + p_ref[1] + hn1_ref[...]
    h1 = jnp.maximum(dinv_ref[...] * s + b1_ref[...], 0.0)
    hn2_ref[...] = (
        jnp.dot(h1, w2_ref[...], preferred_element_type=jnp.float32)
        * dinv_ref[...])


def _tc3_body(p_ref, hn2_ref, dinv_ref, b2_ref, out_ref):
    s = p_ref[0] + p_ref[1] + hn2_ref[...]
    out_ref[...] = dinv_ref[...] * s + b2_ref[...]


@functools.partial(jax.jit, static_argnames=("n_pad", "cpw"))
def _run(x_p, srcp, dstp, W1, b1r, w2p, b2r, n_pad, cpw):
    f32 = jnp.float32
    mesh = plsc.VectorSubcoreMesh(core_axis_name="c", subcore_axis_name="s")
    rows = n_pad // NS

    sc_params = pltpu.CompilerParams(use_tc_tiling_on_sc=False)
    deg_call = pl.kernel(
        _deg_body,
        out_type=jax.ShapeDtypeStruct((NC * n_pad, HIDP), f32),
        mesh=mesh,
        compiler_params=sc_params,
        scratch_types=[
            pltpu.VMEM((IB, CHUNK), jnp.int32),
            pltpu.VMEM((CHUNK, HIDP), f32),
            pltpu.VMEM((rows, HIDP), f32),
            pltpu.VMEM_SHARED((n_pad, HIDP), f32),
            pltpu.SemaphoreType.DMA,
        ],
    )
    edge_call = pl.kernel(
        _edge_body,
        out_type=jax.ShapeDtypeStruct((NC * n_pad, HIDP), f32),
        mesh=mesh,
        compiler_params=sc_params,
        scratch_types=[
            pltpu.VMEM((IB, CHUNK), jnp.int32),
            pltpu.VMEM((IB, CHUNK), jnp.int32),
            pltpu.VMEM((IB, CHUNK, HIDP), f32),
            pltpu.VMEM((rows, HIDP), f32),
            pltpu.VMEM_SHARED((n_pad, HIDP), f32),
            pltpu.SemaphoreType.DMA((IB,)),
            pltpu.SemaphoreType.DMA,
        ],
    )

    degp = deg_call(dstp)
    hn1, dinv = pl.pallas_call(
        _tc1_body,
        out_shape=(jax.ShapeDtypeStruct((n_pad, HIDP), f32),
                   jax.ShapeDtypeStruct((n_pad, HIDP), f32)),
    )(x_p, W1, degp.reshape(NC, n_pad, HIDP))
    p1 = edge_call(srcp, dstp, hn1)
    hn2 = pl.pallas_call(
        _tc2_body,
        out_shape=jax.ShapeDtypeStruct((n_pad, HIDP), f32),
    )(p1.reshape(NC, n_pad, HIDP), hn1, dinv, b1r, w2p)
    p2 = edge_call(srcp, dstp, hn2)
    outf = pl.pallas_call(
        _tc3_body,
        out_shape=jax.ShapeDtypeStruct((n_pad, HIDP), f32),
    )(p2.reshape(NC, n_pad, HIDP), hn2, dinv, b2r)
    return outf


def kernel(x, edge_index, W1, b1, W2, b2):
    f32 = jnp.float32
    n, in_ch = x.shape
    hid = W1.shape[1]
    out_ch = W2.shape[1]
    n_edges = edge_index.shape[1]

    block = NW * CHUNK * IB
    cpw = IB * (-(-n_edges // block))
    e_pad = cpw * NW * CHUNK
    n_pad = -(-n // (NS * 8)) * (NS * 8)
    dummy = n

    ei = edge_index.astype(jnp.int32)
    pad = jnp.full((e_pad - n_edges,), dummy, jnp.int32)
    srcp = jnp.concatenate([ei[0], pad]).reshape(NW, cpw, CHUNK)
    dstp = jnp.concatenate([ei[1], pad]).reshape(NW, cpw, CHUNK)
    x_p = jnp.pad(x.astype(f32), ((0, n_pad - n), (0, 0)))
    b1r = b1.reshape(1, hid).astype(f32)
    w2p = jnp.pad(W2.astype(f32), ((0, 0), (0, HIDP - out_ch)))
    b2r = jnp.pad(b2.reshape(1, out_ch).astype(f32),
                  ((0, 0), (0, HIDP - out_ch)))

    outf = _run(x_p, srcp, dstp, W1.astype(f32), b1r, w2p, b2r,
                n_pad=n_pad, cpw=cpw)
    return outf[:n, :out_ch]


# speedup vs baseline: 52.2298x; 1.5577x over previous
import functools

import jax
import jax.numpy as jnp
from jax import lax
from jax.experimental import pallas as pl
from jax.experimental.pallas import tpu as pltpu
from jax.experimental.pallas import tpu_sc as plsc

NC = 2
NS = 16
NW = NC * NS
LANES = 16
CHUNK = 128
IB = 8
HIDP = 16


def _deg_body(dst_hbm, out_hbm, idx_v, ones_v, stage_v, deg_sh, ssem):
    n_pad = deg_sh.shape[0]
    rows = n_pad // NS
    n_b = dst_hbm.shape[1] // IB
    cid = lax.axis_index("c")
    sid = lax.axis_index("s")
    wid = sid * NC + cid

    def obody(i, carry):
        ones_v[i] = jnp.ones((LANES,), jnp.float32)
        return carry

    lax.fori_loop(0, CHUNK, obody, 0)

    def zbody(i, carry):
        stage_v[i] = jnp.zeros((LANES,), jnp.float32)
        return carry

    lax.fori_loop(0, rows, zbody, 0)
    pltpu.sync_copy(stage_v, deg_sh.at[pl.ds(sid * rows, rows)])
    plsc.subcore_barrier()

    def outer(bi, carry):
        pltpu.sync_copy(dst_hbm.at[wid, pl.ds(bi * IB, IB)], idx_v)
        descs = [
            pltpu.async_copy(ones_v, deg_sh.at[idx_v.at[j]], ssem, add=True)
            for j in range(IB)
        ]
        for d in descs:
            d.wait()
        return carry

    lax.fori_loop(0, n_b, outer, 0)
    plsc.subcore_barrier()
    pltpu.sync_copy(deg_sh.at[pl.ds(sid * rows, rows)], stage_v)
    pltpu.sync_copy(stage_v,
                    out_hbm.at[pl.ds(cid * n_pad + sid * rows, rows)])


def _edge_body(src_hbm, dst_hbm, table_hbm, out_hbm,
               sidx_v, didx_v, rows_v, zbuf_v, table_sh, acc_sh,
               gsems, ssem):
    n_pad = acc_sh.shape[0]
    rows = n_pad // NS
    n_b = src_hbm.shape[1] // IB
    cid = lax.axis_index("c")
    sid = lax.axis_index("s")
    wid = sid * NC + cid
    base = sid * rows
    pieces = [(o, min(CHUNK, rows - o)) for o in range(0, rows, CHUNK)]

    def zbody(i, carry):
        zbuf_v[i] = jnp.zeros((LANES,), jnp.float32)
        return carry

    lax.fori_loop(0, CHUNK, zbody, 0)
    for o, ln in pieces:
        pltpu.sync_copy(zbuf_v.at[pl.ds(0, ln)],
                        acc_sh.at[pl.ds(base + o, ln)])
    tds = [
        pltpu.async_copy(table_hbm.at[pl.ds(base + o, ln)],
                         rows_v.at[k, pl.ds(0, ln)], gsems.at[k])
        for k, (o, ln) in enumerate(pieces)
    ]
    for k, (o, ln) in enumerate(pieces):
        tds[k].wait()
        pltpu.sync_copy(rows_v.at[k, pl.ds(0, ln)],
                        table_sh.at[pl.ds(base + o, ln)])
    plsc.subcore_barrier()

    def outer(bi, carry):
        i0 = pltpu.async_copy(src_hbm.at[wid, pl.ds(bi * IB, IB)], sidx_v,
                              ssem)
        i1 = pltpu.async_copy(dst_hbm.at[wid, pl.ds(bi * IB, IB)], didx_v,
                              ssem)
        i0.wait()
        i1.wait()
        gd = [
            pltpu.async_copy(table_sh.at[sidx_v.at[j]], rows_v.at[j],
                             gsems.at[j])
            for j in range(IB)
        ]
        sd = []
        for j in range(IB):
            gd[j].wait()
            sd.append(
                pltpu.async_copy(rows_v.at[j], acc_sh.at[didx_v.at[j]],
                                 ssem, add=True))
        for d in sd:
            d.wait()
        return carry

    lax.fori_loop(0, n_b, outer, 0)
    plsc.subcore_barrier()
    for k, (o, ln) in enumerate(pieces):
        pltpu.sync_copy(acc_sh.at[pl.ds(base + o, ln)],
                        rows_v.at[k, pl.ds(0, ln)])
        pltpu.sync_copy(rows_v.at[k, pl.ds(0, ln)],
                        out_hbm.at[pl.ds(cid * n_pad + base + o, ln)])


def _tc0_body(x_ref, w_ref, h_ref):
    h_ref[...] = jnp.dot(x_ref[...], w_ref[...],
                         preferred_element_type=jnp.float32)


def _tc1_body(h_ref, degp_ref, hn_ref, dinv_ref):
    n_pad = h_ref.shape[0]
    deg = degp_ref[0:n_pad] + degp_ref[n_pad:2 * n_pad] + 1.0
    dinv = lax.rsqrt(deg)
    hn_ref[...] = h_ref[...] * dinv
    dinv_ref[...] = dinv


def _tc2_body(p_ref, hn1_ref, dinv_ref, b1_ref, w2_ref, hn2_ref):
    n_pad = hn1_ref.shape[0]
    s = p_ref[0:n_pad] + p_ref[n_pad:2 * n_pad] + hn1_ref[...]
    h1 = jnp.maximum(dinv_ref[...] * s + b1_ref[...], 0.0)
    hn2_ref[...] = (
        jnp.dot(h1, w2_ref[...], preferred_element_type=jnp.float32)
        * dinv_ref[...])


def _tc3_body(p_ref, hn2_ref, dinv_ref, b2_ref, out_ref):
    n_pad = hn2_ref.shape[0]
    s = p_ref[0:n_pad] + p_ref[n_pad:2 * n_pad] + hn2_ref[...]
    out_ref[...] = dinv_ref[...] * s + b2_ref[...]


@functools.partial(jax.jit, static_argnames=("n_pad", "cpw"))
def _run(x_p, srcp, dstp, W1, b1r, w2p, b2r, n_pad, cpw):
    f32 = jnp.float32
    mesh = plsc.VectorSubcoreMesh(core_axis_name="c", subcore_axis_name="s")
    rows = n_pad // NS

    sc_params = pltpu.CompilerParams(use_tc_tiling_on_sc=False)
    deg_call = pl.kernel(
        _deg_body,
        out_type=jax.ShapeDtypeStruct((NC * n_pad, HIDP), f32),
        mesh=mesh,
        compiler_params=sc_params,
        scratch_types=[
            pltpu.VMEM((IB, CHUNK), jnp.int32),
            pltpu.VMEM((CHUNK, HIDP), f32),
            pltpu.VMEM((rows, HIDP), f32),
            pltpu.VMEM_SHARED((n_pad, HIDP), f32),
            pltpu.SemaphoreType.DMA,
        ],
    )
    edge_call = pl.kernel(
        _edge_body,
        out_type=jax.ShapeDtypeStruct((NC * n_pad, HIDP), f32),
        mesh=mesh,
        compiler_params=sc_params,
        scratch_types=[
            pltpu.VMEM((IB, CHUNK), jnp.int32),
            pltpu.VMEM((IB, CHUNK), jnp.int32),
            pltpu.VMEM((IB, CHUNK, HIDP), f32),
            pltpu.VMEM((CHUNK, HIDP), f32),
            pltpu.VMEM_SHARED((n_pad, HIDP), f32),
            pltpu.VMEM_SHARED((n_pad, HIDP), f32),
            pltpu.SemaphoreType.DMA((IB,)),
            pltpu.SemaphoreType.DMA,
        ],
    )

    degp = deg_call(dstp)
    h_raw = pl.pallas_call(
        _tc0_body,
        out_shape=jax.ShapeDtypeStruct((n_pad, HIDP), f32),
    )(x_p, W1)
    hn1, dinv = pl.pallas_call(
        _tc1_body,
        out_shape=(jax.ShapeDtypeStruct((n_pad, HIDP), f32),
                   jax.ShapeDtypeStruct((n_pad, HIDP), f32)),
    )(h_raw, degp)
    p1 = edge_call(srcp, dstp, hn1)
    hn2 = pl.pallas_call(
        _tc2_body,
        out_shape=jax.ShapeDtypeStruct((n_pad, HIDP), f32),
    )(p1, hn1, dinv, b1r, w2p)
    p2 = edge_call(srcp, dstp, hn2)
    outf = pl.pallas_call(
        _tc3_body,
        out_shape=jax.ShapeDtypeStruct((n_pad, HIDP), f32),
    )(p2, hn2, dinv, b2r)
    return outf


def kernel(x, edge_index, W1, b1, W2, b2):
    f32 = jnp.float32
    n, in_ch = x.shape
    hid = W1.shape[1]
    out_ch = W2.shape[1]
    n_edges = edge_index.shape[1]

    block = NW * CHUNK * IB
    cpw = IB * (-(-n_edges // block))
    e_pad = cpw * NW * CHUNK
    n_pad = -(-n // (NS * 8)) * (NS * 8)
    dummy = n

    ei = edge_index.astype(jnp.int32)
    pad = jnp.full((e_pad - n_edges,), dummy, jnp.int32)
    srcp = jnp.concatenate([ei[0], pad]).reshape(NW, cpw, CHUNK)
    dstp = jnp.concatenate([ei[1], pad]).reshape(NW, cpw, CHUNK)
    x_p = jnp.pad(x.astype(f32), ((0, n_pad - n), (0, 0)))
    b1r = b1.reshape(1, hid).astype(f32)
    w2p = jnp.pad(W2.astype(f32), ((0, 0), (0, HIDP - out_ch)))
    b2r = jnp.pad(b2.reshape(1, out_ch).astype(f32),
                  ((0, 0), (0, HIDP - out_ch)))

    outf = _run(x_p, srcp, dstp, W1.astype(f32), b1r, w2p, b2r,
                n_pad=n_pad, cpw=cpw)
    return outf[:n, :out_ch]
